# baseline (speedup 1.0000x reference)
import jax
import jax.numpy as jnp
from jax.experimental import pallas as pl
from jax.experimental.pallas import tpu as pltpu

_N = 20000
_K = 100
_PK = 128
_IOU_TH = 0.5


def _ap_from_confusion(C):
    pr_counts = jnp.sum(C, axis=1, keepdims=True)
    gt_counts = jnp.sum(C, axis=0, keepdims=True)
    union = pr_counts + gt_counts - C
    iou = jnp.where(union > 0, C / jnp.maximum(union, 1.0), 0.0)
    pr_present = (pr_counts > 0).astype(jnp.float32)
    gt_present = (gt_counts > 0).astype(jnp.float32)
    tp = jnp.sum((iou >= _IOU_TH).astype(jnp.float32) * pr_present,
                 axis=0, keepdims=True)
    fp = jnp.sum(((iou > 0) & (iou < _IOU_TH)).astype(jnp.float32) * pr_present,
                 axis=0, keepdims=True)
    denom = tp + fp
    precision = jnp.where(denom > 0, tp / jnp.maximum(denom, 1.0), 0.0)
    num = jnp.sum(precision * gt_present, axis=(0, 1), keepdims=True)
    cnt = jnp.sum(gt_present, axis=(0, 1), keepdims=True)
    return num / jnp.maximum(cnt, 1.0)


def _tc_body(pr_ref, gt_ref, out_ref):
    pr = pr_ref[...].astype(jnp.int32)
    gt = gt_ref[...].astype(jnp.int32)
    ids = jax.lax.broadcasted_iota(jnp.int32, (1, _PK), 1)
    a = (pr == ids).astype(jnp.bfloat16)
    b = (gt == ids).astype(jnp.bfloat16)
    C = jax.lax.dot_general(a, b, (((0,), (0,)), ((), ())),
                            preferred_element_type=jnp.float32)
    out_ref[...] = _ap_from_confusion(C)


def kernel(input, target):
    pr = input.reshape(_N, 1)
    gt = target.reshape(_N, 1)
    ap = pl.pallas_call(
        _tc_body,
        out_shape=jax.ShapeDtypeStruct((1, 1), jnp.float32),
    )(pr, gt)
    return ap[0, 0]


# speedup vs baseline: 1.1957x; 1.1957x over previous
import dataclasses
import functools

import jax
import jax.numpy as jnp
from jax import lax
from jax.experimental import pallas as pl
from jax.experimental.pallas import tpu as pltpu
from jax.experimental.pallas import tpu_sc as plsc

_N = 20000
_K = 100
_PK = 128
_NBINS = _K * _PK
_NTILES = 32
_PER_TILE = 640
_NPAD = _NTILES * _PER_TILE
_IOU_TH = 0.5
_LANES = 16

_vector_mesh = plsc.VectorSubcoreMesh(core_axis_name="c", subcore_axis_name="s")

_sc_params = pltpu.CompilerParams()
if "needs_layout_passes" in pltpu.CompilerParams.__dataclass_fields__:
    _sc_params = dataclasses.replace(_sc_params, needs_layout_passes=False)


@functools.partial(
    pl.kernel,
    out_type=jax.ShapeDtypeStruct((_NTILES, _NBINS), jnp.int32),
    mesh=_vector_mesh,
    compiler_params=_sc_params,
    scratch_types=[
        pltpu.VMEM((_PER_TILE,), jnp.float32),
        pltpu.VMEM((_PER_TILE,), jnp.float32),
        pltpu.VMEM((_NBINS,), jnp.int32),
    ],
)
def _sc_hist(pr_hbm, gt_hbm, out_hbm, pr_v, gt_v, hist_v):
    wid = lax.axis_index("s") * 2 + lax.axis_index("c")
    base = wid * _PER_TILE
    pltpu.sync_copy(pr_hbm.at[pl.ds(base, _PER_TILE)], pr_v)
    pltpu.sync_copy(gt_hbm.at[pl.ds(base, _PER_TILE)], gt_v)

    @pl.loop(0, _NBINS, step=_LANES)
    def _zero(j):
        hist_v[pl.ds(j, _LANES)] = jnp.zeros((_LANES,), jnp.int32)

    @pl.loop(0, _PER_TILE, step=_LANES)
    def _acc(j):
        pr = pr_v[pl.ds(j, _LANES)].astype(jnp.int32)
        gt = gt_v[pl.ds(j, _LANES)].astype(jnp.int32)
        pair = pr * _PK + gt
        cnt, last = plsc.scan_count(pair)
        plsc.addupdate_scatter(hist_v, [pair], cnt, mask=last)

    pltpu.sync_copy(hist_v, out_hbm.at[wid])


def _ap_from_confusion(C):
    pr_counts = jnp.sum(C, axis=1, keepdims=True)
    gt_counts = jnp.sum(C, axis=0, keepdims=True)
    union = pr_counts + gt_counts - C
    iou = jnp.where(union > 0, C / jnp.maximum(union, 1.0), 0.0)
    pr_present = (pr_counts > 0).astype(jnp.float32)
    gt_present = (gt_counts > 0).astype(jnp.float32)
    tp = jnp.sum((iou >= _IOU_TH).astype(jnp.float32) * pr_present,
                 axis=0, keepdims=True)
    fp = jnp.sum(((iou > 0) & (iou < _IOU_TH)).astype(jnp.float32) * pr_present,
                 axis=0, keepdims=True)
    denom = tp + fp
    precision = jnp.where(denom > 0, tp / jnp.maximum(denom, 1.0), 0.0)
    num = jnp.sum(precision * gt_present, axis=(0, 1), keepdims=True)
    cnt = jnp.sum(gt_present, axis=(0, 1), keepdims=True)
    return num / jnp.maximum(cnt, 1.0)


def _tc_post(hist_ref, out_ref):
    h = hist_ref[...].reshape(_NTILES, _K, _PK)
    C = jnp.sum(h, axis=0).astype(jnp.float32)
    col = lax.broadcasted_iota(jnp.int32, (1, _PK), 1)
    C = jnp.where(col < _K, C, 0.0)
    out_ref[...] = _ap_from_confusion(C)


def kernel(input, target):
    pr = jnp.concatenate([input, jnp.zeros((_NPAD - _N,), jnp.float32)])
    gt = jnp.concatenate([target, jnp.full((_NPAD - _N,), float(_K), jnp.float32)])
    hist = _sc_hist(pr, gt)
    ap = pl.pallas_call(
        _tc_post,
        out_shape=jax.ShapeDtypeStruct((1, 1), jnp.float32),
    )(hist)
    return ap[0, 0]


# speedup vs baseline: 1.2184x; 1.0190x over previous
import dataclasses
import functools

import jax
import jax.numpy as jnp
from jax import lax
from jax.experimental import pallas as pl
from jax.experimental.pallas import tpu as pltpu
from jax.experimental.pallas import tpu_sc as plsc

_N = 20000
_K = 100
_PK = 128
_NBINS = _K * _PK
_NTILES = 32
_PER_TILE = 640
_NPAD = _NTILES * _PER_TILE
_IOU_TH = 0.5
_LANES = 16

_vector_mesh = plsc.VectorSubcoreMesh(core_axis_name="c", subcore_axis_name="s")

_sc_params = pltpu.CompilerParams()
if "needs_layout_passes" in pltpu.CompilerParams.__dataclass_fields__:
    _sc_params = dataclasses.replace(_sc_params, needs_layout_passes=False)


@functools.partial(
    pl.kernel,
    out_type=jax.ShapeDtypeStruct((_NTILES, _NBINS), jnp.int32),
    mesh=_vector_mesh,
    compiler_params=_sc_params,
    scratch_types=[
        pltpu.VMEM((_PER_TILE,), jnp.float32),
        pltpu.VMEM((_PER_TILE,), jnp.float32),
        pltpu.VMEM((_NBINS,), jnp.int32),
        pltpu.SemaphoreType.DMA,
    ],
)
def _sc_hist(pr_hbm, gt_hbm, zeros_hbm, out_hbm, pr_v, gt_v, hist_v, zsem):
    wid = lax.axis_index("s") * 2 + lax.axis_index("c")
    base = wid * _PER_TILE
    zero_cp = pltpu.async_copy(zeros_hbm, hist_v, zsem)
    pltpu.sync_copy(pr_hbm.at[pl.ds(base, _PER_TILE)], pr_v)
    pltpu.sync_copy(gt_hbm.at[pl.ds(base, _PER_TILE)], gt_v)
    zero_cp.wait()

    @pl.loop(0, _PER_TILE, step=4 * _LANES)
    def _acc(j):
        for k in range(4):
            pr = pr_v[pl.ds(j + k * _LANES, _LANES)]
            gt = gt_v[pl.ds(j + k * _LANES, _LANES)]
            pair = (pr * float(_PK) + gt).astype(jnp.int32)
            cnt, last = plsc.scan_count(pair)
            plsc.addupdate_scatter(hist_v, [pair], cnt, mask=last)

    pltpu.sync_copy(hist_v, out_hbm.at[wid])


def _ap_from_confusion(C):
    pr_counts = jnp.sum(C, axis=1, keepdims=True)
    gt_counts = jnp.sum(C, axis=0, keepdims=True)
    union = pr_counts + gt_counts - C
    iou = jnp.where(union > 0, C / jnp.maximum(union, 1.0), 0.0)
    pr_present = (pr_counts > 0).astype(jnp.float32)
    gt_present = (gt_counts > 0).astype(jnp.float32)
    tp = jnp.sum((iou >= _IOU_TH).astype(jnp.float32) * pr_present,
                 axis=0, keepdims=True)
    fp = jnp.sum(((iou > 0) & (iou < _IOU_TH)).astype(jnp.float32) * pr_present,
                 axis=0, keepdims=True)
    denom = tp + fp
    precision = jnp.where(denom > 0, tp / jnp.maximum(denom, 1.0), 0.0)
    num = jnp.sum(precision * gt_present, axis=(0, 1), keepdims=True)
    cnt = jnp.sum(gt_present, axis=(0, 1), keepdims=True)
    return num / jnp.maximum(cnt, 1.0)


def _tc_post(hist_ref, out_ref):
    h = hist_ref[...].reshape(_NTILES, _K, _PK)
    C = jnp.sum(h, axis=0).astype(jnp.float32)
    col = lax.broadcasted_iota(jnp.int32, (1, _PK), 1)
    C = jnp.where(col < _K, C, 0.0)
    out_ref[...] = _ap_from_confusion(C)


def kernel(input, target):
    pr = jnp.concatenate([input, jnp.zeros((_NPAD - _N,), jnp.float32)])
    gt = jnp.concatenate([target, jnp.full((_NPAD - _N,), float(_K), jnp.float32)])
    zeros = jnp.zeros((_NBINS,), jnp.int32)
    hist = _sc_hist(pr, gt, zeros)
    ap = pl.pallas_call(
        _tc_post,
        out_shape=jax.ShapeDtypeStruct((1, 1), jnp.float32),
    )(hist)
    return ap[0, 0]


# speedup vs baseline: 1.2849x; 1.0545x over previous
import dataclasses
import functools

import jax
import jax.numpy as jnp
from jax import lax
from jax.experimental import pallas as pl
from jax.experimental.pallas import tpu as pltpu
from jax.experimental.pallas import tpu_sc as plsc

_N = 20000
_K = 100
_PK = 128
_NBINS = _K * _PK
_NCORES = 2
_NSUB = 16
_NTILES = _NCORES * _NSUB
_PER_TILE = 640
_TAIL = _N - (_NTILES - 1) * _PER_TILE
_IOU_TH = 0.5
_LANES = 16
_UNROLL = 2 * _LANES

_vector_mesh = plsc.VectorSubcoreMesh(core_axis_name="c", subcore_axis_name="s")

_sc_params = pltpu.CompilerParams()
if "needs_layout_passes" in pltpu.CompilerParams.__dataclass_fields__:
    _sc_params = dataclasses.replace(_sc_params, needs_layout_passes=False)


@functools.partial(
    pl.kernel,
    out_type=jax.ShapeDtypeStruct((_NCORES, 1, _NBINS), jnp.int32),
    mesh=_vector_mesh,
    compiler_params=_sc_params,
    scratch_types=[
        pltpu.VMEM((_PER_TILE,), jnp.float32),
        pltpu.VMEM((_PER_TILE,), jnp.float32),
        pltpu.VMEM((1, _NBINS), jnp.int32),
        pltpu.VMEM((1,), jnp.int32),
        pltpu.VMEM_SHARED((1, _NBINS), jnp.int32),
        pltpu.SemaphoreType.DMA,
    ],
)
def _sc_hist(pr_hbm, gt_hbm, zeros_hbm, out_hbm, pr_v, gt_v, hist_v, row0_v,
             shared, zsem):
    cid = lax.axis_index("c")
    sid = lax.axis_index("s")
    wid = sid * _NCORES + cid
    base = wid * _PER_TILE

    zero_cp = pltpu.async_copy(zeros_hbm, hist_v, zsem)

    @pl.when(sid == 0)
    def _zero_shared():
        pltpu.sync_copy(zeros_hbm, shared)

    @pl.when(wid < _NTILES - 1)
    def _load_full():
        pltpu.sync_copy(pr_hbm.at[pl.ds(base, _PER_TILE)], pr_v)
        pltpu.sync_copy(gt_hbm.at[pl.ds(base, _PER_TILE)], gt_v)

    @pl.when(wid == _NTILES - 1)
    def _load_tail():
        pltpu.sync_copy(pr_hbm.at[pl.ds(base, _TAIL)], pr_v.at[pl.ds(0, _TAIL)])
        pltpu.sync_copy(gt_hbm.at[pl.ds(base, _TAIL)], gt_v.at[pl.ds(0, _TAIL)])

    lane0 = lax.iota(jnp.int32, _LANES) == 0
    zeros16 = jnp.zeros((_LANES,), jnp.int32)
    plsc.store_scatter(row0_v, [zeros16], zeros16, mask=lane0)

    zero_cp.wait()

    npts = jnp.where(wid == _NTILES - 1, _TAIL, _PER_TILE)

    @pl.loop(0, npts, step=_UNROLL)
    def _acc(j):
        for k in range(_UNROLL // _LANES):
            pr = pr_v[pl.ds(j + k * _LANES, _LANES)]
            gt = gt_v[pl.ds(j + k * _LANES, _LANES)]
            pair = (pr * float(_PK) + gt).astype(jnp.int32)
            cnt, last = plsc.scan_count(pair)
            plsc.addupdate_scatter(hist_v, [zeros16, pair], cnt, mask=last)

    plsc.subcore_barrier()
    pltpu.sync_copy(hist_v, shared.at[row0_v], add=True)
    plsc.subcore_barrier()

    @pl.when(sid == 0)
    def _flush():
        pltpu.sync_copy(shared, out_hbm.at[cid])


def _ap_from_confusion(C):
    pr_counts = jnp.sum(C, axis=1, keepdims=True)
    gt_counts = jnp.sum(C, axis=0, keepdims=True)
    union = pr_counts + gt_counts - C
    iou = jnp.where(union > 0, C / jnp.maximum(union, 1.0), 0.0)
    pr_present = (pr_counts > 0).astype(jnp.float32)
    gt_present = (gt_counts > 0).astype(jnp.float32)
    tp = jnp.sum((iou >= _IOU_TH).astype(jnp.float32) * pr_present,
                 axis=0, keepdims=True)
    fp = jnp.sum(((iou > 0) & (iou < _IOU_TH)).astype(jnp.float32) * pr_present,
                 axis=0, keepdims=True)
    denom = tp + fp
    precision = jnp.where(denom > 0, tp / jnp.maximum(denom, 1.0), 0.0)
    num = jnp.sum(precision * gt_present, axis=(0, 1), keepdims=True)
    cnt = jnp.sum(gt_present, axis=(0, 1), keepdims=True)
    return num / jnp.maximum(cnt, 1.0)


def _tc_post(hist_ref, out_ref):
    h = hist_ref[...].reshape(_NCORES, _K, _PK)
    C = jnp.sum(h, axis=0).astype(jnp.float32)
    out_ref[...] = _ap_from_confusion(C)


def kernel(input, target):
    zeros = jnp.zeros((1, _NBINS), jnp.int32)
    hist = _sc_hist(input, target, zeros)
    ap = pl.pallas_call(
        _tc_post,
        out_shape=jax.ShapeDtypeStruct((1, 1), jnp.float32),
    )(hist)
    return ap[0, 0]


# speedup vs baseline: 1.3791x; 1.0733x over previous
import dataclasses
import functools

import jax
import jax.numpy as jnp
from jax import lax
from jax.experimental import pallas as pl
from jax.experimental.pallas import tpu as pltpu
from jax.experimental.pallas import tpu_sc as plsc

_N = 20000
_K = 100
_PK = 128
_NBINS = _K * _PK
_DUMMY = _NBINS
_SBINS = _NBINS + 16
_NCORES = 2
_NSUB = 16
_NTILES = _NCORES * _NSUB
_PER_TILE = 640
_TAIL = _N - (_NTILES - 1) * _PER_TILE
_IOU_TH = 0.5
_LANES = 16
_CHUNK = 128
_NCHUNKS = _PER_TILE // _CHUNK

_vector_mesh = plsc.VectorSubcoreMesh(core_axis_name="c", subcore_axis_name="s")

_sc_params = pltpu.CompilerParams()
if "needs_layout_passes" in pltpu.CompilerParams.__dataclass_fields__:
    _sc_params = dataclasses.replace(_sc_params, needs_layout_passes=False)


@functools.partial(
    pl.kernel,
    out_type=jax.ShapeDtypeStruct((_NCORES, _NBINS), jnp.int32),
    mesh=_vector_mesh,
    compiler_params=_sc_params,
    scratch_types=[
        pltpu.VMEM((_PER_TILE,), jnp.float32),
        pltpu.VMEM((_PER_TILE,), jnp.float32),
        [pltpu.VMEM((_CHUNK,), jnp.int32) for _ in range(_NCHUNKS)],
        pltpu.VMEM((_CHUNK,), jnp.int32),
        pltpu.VMEM_SHARED((_SBINS,), jnp.int32),
    ],
)
def _sc_hist(pr_hbm, gt_hbm, zeros_hbm, out_hbm, pr_v, gt_v, idx_refs, ones_v,
             shared):
    cid = lax.axis_index("c")
    sid = lax.axis_index("s")
    wid = sid * _NCORES + cid
    base = wid * _PER_TILE

    @pl.when(sid == 0)
    def _zero_shared():
        pltpu.sync_copy(zeros_hbm, shared)

    @pl.when(wid < _NTILES - 1)
    def _load_full():
        pltpu.sync_copy(pr_hbm.at[pl.ds(base, _PER_TILE)], pr_v)
        pltpu.sync_copy(gt_hbm.at[pl.ds(base, _PER_TILE)], gt_v)

    @pl.when(wid == _NTILES - 1)
    def _load_tail():
        pltpu.sync_copy(pr_hbm.at[pl.ds(base, _TAIL)], pr_v.at[pl.ds(0, _TAIL)])
        pltpu.sync_copy(gt_hbm.at[pl.ds(base, _TAIL)], gt_v.at[pl.ds(0, _TAIL)])

    npts = jnp.where(wid == _NTILES - 1, _TAIL, _PER_TILE)
    lane = lax.iota(jnp.int32, _LANES)
    one16 = jnp.ones((_LANES,), jnp.int32)

    for v in range(_CHUNK // _LANES):
        ones_v[pl.ds(v * _LANES, _LANES)] = one16

    for c in range(_NCHUNKS):
        for v in range(_CHUNK // _LANES):
            i = c * _CHUNK + v * _LANES
            pr = pr_v[pl.ds(i, _LANES)]
            gt = gt_v[pl.ds(i, _LANES)]
            pair = (pr * float(_PK) + gt).astype(jnp.int32)
            valid = (lane + i) < npts
            idx_refs[c][pl.ds(v * _LANES, _LANES)] = jnp.where(
                valid, pair, _DUMMY)

    plsc.subcore_barrier()
    for c in range(_NCHUNKS):
        pltpu.sync_copy(ones_v, shared.at[idx_refs[c]], add=True)
    plsc.subcore_barrier()

    @pl.when(sid == 0)
    def _flush():
        pltpu.sync_copy(shared.at[pl.ds(0, _NBINS)], out_hbm.at[cid])


def _ap_from_confusion(C):
    pr_counts = jnp.sum(C, axis=1, keepdims=True)
    gt_counts = jnp.sum(C, axis=0, keepdims=True)
    union = pr_counts + gt_counts - C
    iou = jnp.where(union > 0, C / jnp.maximum(union, 1.0), 0.0)
    pr_present = (pr_counts > 0).astype(jnp.float32)
    gt_present = (gt_counts > 0).astype(jnp.float32)
    tp = jnp.sum((iou >= _IOU_TH).astype(jnp.float32) * pr_present,
                 axis=0, keepdims=True)
    fp = jnp.sum(((iou > 0) & (iou < _IOU_TH)).astype(jnp.float32) * pr_present,
                 axis=0, keepdims=True)
    denom = tp + fp
    precision = jnp.where(denom > 0, tp / jnp.maximum(denom, 1.0), 0.0)
    num = jnp.sum(precision * gt_present, axis=(0, 1), keepdims=True)
    cnt = jnp.sum(gt_present, axis=(0, 1), keepdims=True)
    return num / jnp.maximum(cnt, 1.0)


def _tc_post(hist_ref, out_ref):
    h = hist_ref[...].reshape(_NCORES, _K, _PK)
    C = jnp.sum(h, axis=0).astype(jnp.float32)
    out_ref[...] = _ap_from_confusion(C)


def kernel(input, target):
    zeros = jnp.zeros((_SBINS,), jnp.int32)
    hist = _sc_hist(input, target, zeros)
    ap = pl.pallas_call(
        _tc_post,
        out_shape=jax.ShapeDtypeStruct((1, 1), jnp.float32),
    )(hist)
    return ap[0, 0]


# speedup vs baseline: 1.4194x; 1.0292x over previous
import dataclasses
import functools

import jax
import jax.numpy as jnp
from jax import lax
from jax.experimental import pallas as pl
from jax.experimental.pallas import tpu as pltpu
from jax.experimental.pallas import tpu_sc as plsc

_N = 20000
_K = 100
_PK = 128
_NBINS = _K * _PK
_DUMMY = _NBINS
_NCORES = 2
_NSUB = 16
_NTILES = _NCORES * _NSUB
_SLAB = 816
_SBINS = _NSUB * _SLAB
_FLUSH = _NBINS // _NSUB
_PER_TILE = 640
_TAIL = _N - (_NTILES - 1) * _PER_TILE
_IOU_TH = 0.5
_LANES = 16
_CHUNK = 128
_NCHUNKS = _PER_TILE // _CHUNK

_vector_mesh = plsc.VectorSubcoreMesh(core_axis_name="c", subcore_axis_name="s")

_sc_params = pltpu.CompilerParams()
if "needs_layout_passes" in pltpu.CompilerParams.__dataclass_fields__:
    _sc_params = dataclasses.replace(_sc_params, needs_layout_passes=False)


@functools.partial(
    pl.kernel,
    out_type=jax.ShapeDtypeStruct((_NCORES, _NBINS), jnp.int32),
    mesh=_vector_mesh,
    compiler_params=_sc_params,
    scratch_types=[
        pltpu.VMEM((_PER_TILE,), jnp.float32),
        pltpu.VMEM((_PER_TILE,), jnp.float32),
        [pltpu.VMEM((_CHUNK,), jnp.int32) for _ in range(_NCHUNKS)],
        pltpu.VMEM((_CHUNK,), jnp.int32),
        pltpu.VMEM((_SLAB,), jnp.int32),
        pltpu.VMEM_SHARED((_SBINS,), jnp.int32),
        pltpu.SemaphoreType.DMA,
    ],
)
def _sc_hist(pr_hbm, gt_hbm, out_hbm, pr_v, gt_v, idx_refs, ones_v, zslab_v,
             shared, add_sem):
    cid = lax.axis_index("c")
    sid = lax.axis_index("s")
    wid = sid * _NCORES + cid
    base = wid * _PER_TILE

    @pl.when(wid < _NTILES - 1)
    def _load_full():
        pltpu.sync_copy(pr_hbm.at[pl.ds(base, _PER_TILE)], pr_v)
        pltpu.sync_copy(gt_hbm.at[pl.ds(base, _PER_TILE)], gt_v)

    @pl.when(wid == _NTILES - 1)
    def _load_tail():
        pltpu.sync_copy(pr_hbm.at[pl.ds(base, _TAIL)], pr_v.at[pl.ds(0, _TAIL)])
        pltpu.sync_copy(gt_hbm.at[pl.ds(base, _TAIL)], gt_v.at[pl.ds(0, _TAIL)])

    zero16 = jnp.zeros((_LANES,), jnp.int32)
    one16 = jnp.ones((_LANES,), jnp.int32)
    for v in range(_SLAB // _LANES):
        zslab_v[pl.ds(v * _LANES, _LANES)] = zero16
    for v in range(_CHUNK // _LANES):
        ones_v[pl.ds(v * _LANES, _LANES)] = one16
    pltpu.sync_copy(zslab_v, shared.at[pl.ds(sid * _SLAB, _SLAB)])

    npts = jnp.where(wid == _NTILES - 1, _TAIL, _PER_TILE)
    lane = lax.iota(jnp.int32, _LANES)
    for c in range(_NCHUNKS):
        for v in range(_CHUNK // _LANES):
            i = c * _CHUNK + v * _LANES
            pr = pr_v[pl.ds(i, _LANES)]
            gt = gt_v[pl.ds(i, _LANES)]
            pair = (pr * float(_PK) + gt).astype(jnp.int32)
            valid = (lane + i) < npts
            idx_refs[c][pl.ds(v * _LANES, _LANES)] = jnp.where(
                valid, pair, _DUMMY)

    plsc.subcore_barrier()
    add_cps = [
        pltpu.async_copy(ones_v, shared.at[idx_refs[c]], add=True,
                         sem=add_sem)
        for c in range(_NCHUNKS)
    ]
    for cp in add_cps:
        cp.wait()
    plsc.subcore_barrier()

    @pl.when(sid == 0)
    def _flush():
        pltpu.sync_copy(shared.at[pl.ds(0, _NBINS)], out_hbm.at[cid])


def _ap_from_confusion(C):
    pr_counts = jnp.sum(C, axis=1, keepdims=True)
    gt_counts = jnp.sum(C, axis=0, keepdims=True)
    union = pr_counts + gt_counts - C
    iou = jnp.where(union > 0, C / jnp.maximum(union, 1.0), 0.0)
    pr_present = (pr_counts > 0).astype(jnp.float32)
    gt_present = (gt_counts > 0).astype(jnp.float32)
    tp = jnp.sum((iou >= _IOU_TH).astype(jnp.float32) * pr_present,
                 axis=0, keepdims=True)
    fp = jnp.sum(((iou > 0) & (iou < _IOU_TH)).astype(jnp.float32) * pr_present,
                 axis=0, keepdims=True)
    denom = tp + fp
    precision = jnp.where(denom > 0, tp / jnp.maximum(denom, 1.0), 0.0)
    num = jnp.sum(precision * gt_present, axis=(0, 1), keepdims=True)
    cnt = jnp.sum(gt_present, axis=(0, 1), keepdims=True)
    return num / jnp.maximum(cnt, 1.0)


def _tc_post(hist_ref, out_ref):
    h = hist_ref[...].reshape(_NCORES, _K, _PK)
    C = jnp.sum(h, axis=0).astype(jnp.float32)
    out_ref[...] = _ap_from_confusion(C)


def kernel(input, target):
    hist = _sc_hist(input, target)
    ap = pl.pallas_call(
        _tc_post,
        out_shape=jax.ShapeDtypeStruct((1, 1), jnp.float32),
    )(hist)
    return ap[0, 0]


# speedup vs baseline: 1.4546x; 1.0248x over previous
import dataclasses
import functools

import jax
import jax.numpy as jnp
from jax import lax
from jax.experimental import pallas as pl
from jax.experimental.pallas import tpu as pltpu
from jax.experimental.pallas import tpu_sc as plsc

_N = 20000
_K = 100
_PK = 128
_NBINS = _K * _PK
_DUMMY = _NBINS
_NCORES = 2
_NSUB = 16
_NTILES = _NCORES * _NSUB
_SLAB = 816
_SBINS = _NSUB * _SLAB
_FLUSH = _NBINS // _NSUB
_PER_TILE = 640
_TAIL = _N - (_NTILES - 1) * _PER_TILE
_IOU_TH = 0.5
_LANES = 16
_CHUNK = 128
_NCHUNKS = _PER_TILE // _CHUNK

_vector_mesh = plsc.VectorSubcoreMesh(core_axis_name="c", subcore_axis_name="s")

_sc_params = pltpu.CompilerParams()
if "needs_layout_passes" in pltpu.CompilerParams.__dataclass_fields__:
    _sc_params = dataclasses.replace(_sc_params, needs_layout_passes=False)


@functools.partial(
    pl.kernel,
    out_type=jax.ShapeDtypeStruct((_NCORES, _NBINS), jnp.int32),
    mesh=_vector_mesh,
    compiler_params=_sc_params,
    scratch_types=[
        pltpu.VMEM((_PER_TILE,), jnp.float32),
        pltpu.VMEM((_PER_TILE,), jnp.float32),
        [pltpu.VMEM((_CHUNK,), jnp.int32) for _ in range(_NCHUNKS)],
        pltpu.VMEM((_CHUNK,), jnp.int32),
        pltpu.VMEM((_SLAB,), jnp.int32),
        pltpu.VMEM_SHARED((_SBINS,), jnp.int32),
        pltpu.SemaphoreType.DMA,
        pltpu.SemaphoreType.DMA,
        pltpu.SemaphoreType.DMA,
    ],
)
def _sc_hist(pr_hbm, gt_hbm, out_hbm, pr_v, gt_v, idx_refs, ones_v, zslab_v,
             shared, add_sem, in_sem, z_sem):
    cid = lax.axis_index("c")
    sid = lax.axis_index("s")
    wid = sid * _NCORES + cid
    base = wid * _PER_TILE

    @pl.when(wid < _NTILES - 1)
    def _load_full():
        pltpu.async_copy(pr_hbm.at[pl.ds(base, _PER_TILE)], pr_v, in_sem)
        pltpu.async_copy(gt_hbm.at[pl.ds(base, _PER_TILE)], gt_v, in_sem)

    @pl.when(wid == _NTILES - 1)
    def _load_tail():
        pltpu.async_copy(pr_hbm.at[pl.ds(base, _TAIL)],
                         pr_v.at[pl.ds(0, _TAIL)], in_sem)
        pltpu.async_copy(gt_hbm.at[pl.ds(base, _TAIL)],
                         gt_v.at[pl.ds(0, _TAIL)], in_sem)

    zero16 = jnp.zeros((_LANES,), jnp.int32)
    one16 = jnp.ones((_LANES,), jnp.int32)
    for v in range(_SLAB // _LANES):
        zslab_v[pl.ds(v * _LANES, _LANES)] = zero16
    for v in range(_CHUNK // _LANES):
        ones_v[pl.ds(v * _LANES, _LANES)] = one16
    pltpu.async_copy(zslab_v, shared.at[pl.ds(sid * _SLAB, _SLAB)], z_sem)

    @pl.when(wid < _NTILES - 1)
    def _wait_full():
        pltpu.make_async_copy(
            pr_hbm.at[pl.ds(base, _PER_TILE)], pr_v, in_sem).wait()
        pltpu.make_async_copy(
            gt_hbm.at[pl.ds(base, _PER_TILE)], gt_v, in_sem).wait()

    @pl.when(wid == _NTILES - 1)
    def _wait_tail():
        pltpu.make_async_copy(
            pr_hbm.at[pl.ds(base, _TAIL)], pr_v.at[pl.ds(0, _TAIL)],
            in_sem).wait()
        pltpu.make_async_copy(
            gt_hbm.at[pl.ds(base, _TAIL)], gt_v.at[pl.ds(0, _TAIL)],
            in_sem).wait()

    npts = jnp.where(wid == _NTILES - 1, _TAIL, _PER_TILE)
    lane = lax.iota(jnp.int32, _LANES)
    for c in range(_NCHUNKS):
        for v in range(_CHUNK // _LANES):
            i = c * _CHUNK + v * _LANES
            pr = pr_v[pl.ds(i, _LANES)]
            gt = gt_v[pl.ds(i, _LANES)]
            pair = (pr * float(_PK) + gt).astype(jnp.int32)
            valid = (lane + i) < npts
            idx_refs[c][pl.ds(v * _LANES, _LANES)] = jnp.where(
                valid, pair, _DUMMY)

    pltpu.make_async_copy(
        zslab_v, shared.at[pl.ds(sid * _SLAB, _SLAB)], z_sem).wait()
    plsc.subcore_barrier()
    add_cps = [
        pltpu.async_copy(ones_v, shared.at[idx_refs[c]], add=True,
                         sem=add_sem)
        for c in range(_NCHUNKS)
    ]
    for cp in add_cps:
        cp.wait()
    plsc.subcore_barrier()

    @pl.when(sid == 0)
    def _flush():
        pltpu.sync_copy(shared.at[pl.ds(0, _NBINS)], out_hbm.at[cid])


def _ap_from_confusion(C):
    pr_counts = jnp.sum(C, axis=1, keepdims=True)
    gt_counts = jnp.sum(C, axis=0, keepdims=True)
    union = pr_counts + gt_counts - C
    iou = jnp.where(union > 0, C / jnp.maximum(union, 1.0), 0.0)
    pr_present = (pr_counts > 0).astype(jnp.float32)
    gt_present = (gt_counts > 0).astype(jnp.float32)
    tp = jnp.sum((iou >= _IOU_TH).astype(jnp.float32) * pr_present,
                 axis=0, keepdims=True)
    fp = jnp.sum(((iou > 0) & (iou < _IOU_TH)).astype(jnp.float32) * pr_present,
                 axis=0, keepdims=True)
    denom = tp + fp
    precision = jnp.where(denom > 0, tp / jnp.maximum(denom, 1.0), 0.0)
    num = jnp.sum(precision * gt_present, axis=(0, 1), keepdims=True)
    cnt = jnp.sum(gt_present, axis=(0, 1), keepdims=True)
    return num / jnp.maximum(cnt, 1.0)


def _tc_post(hist_ref, out_ref):
    h = hist_ref[...].reshape(_NCORES, _K, _PK)
    C = jnp.sum(h, axis=0).astype(jnp.float32)
    out_ref[...] = _ap_from_confusion(C)


def kernel(input, target):
    hist = _sc_hist(input, target)
    ap = pl.pallas_call(
        _tc_post,
        out_shape=jax.ShapeDtypeStruct((1, 1), jnp.float32),
    )(hist)
    return ap[0, 0]
